# baseline (speedup 1.0000x reference)
import functools

import jax
import jax.numpy as jnp
from jax import lax
from jax.experimental import pallas as pl
from jax.experimental.pallas import tpu as pltpu
from jax.experimental.pallas import tpu_sc as plsc

D = 128
B = 16384
NE = 100000
NC = 2
NS = 16
NW = NC * NS
BPW = B // NW
CHUNK = 128
NCHUNK = BPW // CHUNK
ER = 8192
ERW = ER // NW
ERH = ERW // 2
ETC = NE - ER
LAMBDA_R = 0.05
LAMBDA_E = 0.01
SGNBIT = -2147483648


def _perm(x, idx):
    return jnp.take_along_axis(x, idx, axis=0)


def _rsqrt_nr(x):
    i = plsc.bitcast(x, jnp.int32)
    y = plsc.bitcast(jnp.int32(0x5F3759DF) - (i >> 1), jnp.float32)
    hx = 0.5 * x
    for _ in range(2):
        y = y * (1.5 - hx * y * y)
    return y


_sc_mesh = plsc.VectorSubcoreMesh(core_axis_name="c", subcore_axis_name="s")


@functools.partial(
    pl.kernel,
    mesh=_sc_mesh,
    compiler_params=pltpu.CompilerParams(needs_layout_passes=False),
    out_type=(jax.ShapeDtypeStruct((B,), jnp.float32),
              jax.ShapeDtypeStruct((NW * 16,), jnp.float32),
              jax.ShapeDtypeStruct((NW * 16,), jnp.float32)),
    scratch_types=[
        pltpu.VMEM((CHUNK,), jnp.int32),
        pltpu.VMEM((CHUNK,), jnp.int32),
        pltpu.VMEM((CHUNK,), jnp.int32),
        pltpu.VMEM((CHUNK, D), jnp.float32),
        pltpu.VMEM((CHUNK, D), jnp.float32),
        pltpu.VMEM((CHUNK, D), jnp.float32),
        pltpu.VMEM((CHUNK + 16,), jnp.float32),
        pltpu.SemaphoreType.DMA,
    ],
)
def _phi_sc(ent, rel, hidx, ridx, tidx, phi_out, sce_out, scr_out,
            hI, rI, tI, Hv, Rv, Tv, phiv, sem):
    wid = lax.axis_index("s") * NC + lax.axis_index("c")
    base = wid * BPW
    lanes = lax.iota(jnp.int32, 16)
    comp = lanes & 3
    px1 = lanes ^ 1
    px2 = lanes ^ 2
    px3 = lanes ^ 3
    px4 = lanes ^ 4
    px8 = lanes ^ 8
    grp = lanes & (-4)
    pb0 = grp
    pb1 = grp | 1
    pb2 = grp | 2
    pb3 = grp | 3
    zero = jnp.zeros((16,), jnp.int32)
    s0 = jnp.where(comp != 0, SGNBIT, zero)
    s1 = jnp.where(comp == 3, SGNBIT, zero)
    s2 = jnp.where(comp == 1, SGNBIT, zero)
    s3 = jnp.where(comp == 2, SGNBIT, zero)
    lane0 = lanes == 0

    def row_phi(row):
        acc = jnp.zeros((16,), jnp.float32)
        for v in range(D // 16):
            h = Hv[row, pl.ds(v * 16, 16)]
            r = Rv[row, pl.ds(v * 16, 16)]
            t = Tv[row, pl.ds(v * 16, 16)]
            r2 = r * r
            na = r2 + _perm(r2, px1)
            n = na + _perm(na, px2)
            rinv = _rsqrt_nr(n)
            hh = h * rinv
            ti = plsc.bitcast(t, jnp.int32)
            t0 = plsc.bitcast(_perm(ti, pb0) ^ s0, jnp.float32)
            t1 = plsc.bitcast(_perm(ti, pb1) ^ s1, jnp.float32)
            t2 = plsc.bitcast(_perm(ti, pb2) ^ s2, jnp.float32)
            t3 = plsc.bitcast(_perm(ti, pb3) ^ s3, jnp.float32)
            g = (r * t0 + _perm(r, px1) * t1
                 + _perm(r, px2) * t2 + _perm(r, px3) * t3)
            acc = acc + hh * g
        acc = acc + _perm(acc, px1)
        acc = acc + _perm(acc, px2)
        acc = acc + _perm(acc, px4)
        acc = acc + _perm(acc, px8)
        return acc

    def chunk_body(ci, _):
        off = base + ci * CHUNK
        iH = pltpu.async_copy(hidx.at[pl.ds(off, CHUNK)], hI, sem)
        iR = pltpu.async_copy(ridx.at[pl.ds(off, CHUNK)], rI, sem)
        iT = pltpu.async_copy(tidx.at[pl.ds(off, CHUNK)], tI, sem)
        iH.wait()
        iR.wait()
        iT.wait()
        cH = pltpu.async_copy(ent.at[hI], Hv, sem)
        cR = pltpu.async_copy(rel.at[rI], Rv, sem)
        cT = pltpu.async_copy(ent.at[tI], Tv, sem)
        cH.wait()
        cR.wait()
        cT.wait()

        def row_body(i, _):
            row = i * 4
            ps = [row_phi(row + j) for j in range(4)]
            for j in range(4):
                plsc.store_compressed(phiv.at[pl.ds(row + j, 16)], ps[j],
                                      mask=lane0)
            return _

        lax.fori_loop(0, CHUNK // 4, row_body, 0)
        pltpu.sync_copy(phiv.at[pl.ds(0, CHUNK)],
                        phi_out.at[pl.ds(off, CHUNK)])
        return _

    lax.fori_loop(0, NCHUNK, chunk_body, 0)

    def buf_ssq(buf, a):
        def nrow(i, a2):
            for v in range(D // 16):
                x = buf[i, pl.ds(v * 16, 16)]
                a2 = a2 + x * x
            return a2
        return lax.fori_loop(0, ERH, nrow, a)

    nbase = ETC + wid * ERW
    z16 = jnp.zeros((16,), jnp.float32)
    c0 = pltpu.async_copy(ent.at[pl.ds(nbase, ERH), :], Hv, sem)
    c1 = pltpu.async_copy(ent.at[pl.ds(nbase + ERH, ERH), :], Rv, sem)
    c2 = pltpu.async_copy(rel.at[pl.ds(nbase, ERH), :], Tv, sem)
    c0.wait()
    essq = buf_ssq(Hv, z16)
    c3 = pltpu.async_copy(rel.at[pl.ds(nbase + ERH, ERH), :], Hv, sem)
    c1.wait()
    essq = buf_ssq(Rv, essq)
    c2.wait()
    rssq = buf_ssq(Tv, z16)
    c3.wait()
    rssq = buf_ssq(Hv, rssq)
    phiv[pl.ds(0, 16)] = essq
    pltpu.sync_copy(phiv.at[pl.ds(0, 16)], sce_out.at[pl.ds(wid * 16, 16)])
    phiv[pl.ds(0, 16)] = rssq
    pltpu.sync_copy(phiv.at[pl.ds(0, 16)], scr_out.at[pl.ds(wid * 16, 16)])


RB = 2416
NBLK = ETC // RB // 2


def _ssq_body(e1_ref, e2_ref, r1_ref, r2_ref, o_ref, acc):
    i = pl.program_id(0)

    @pl.when(i == 0)
    def _():
        acc[0] = 0.0
        acc[1] = 0.0

    e1 = e1_ref[...]
    e2 = e2_ref[...]
    r1 = r1_ref[...]
    r2 = r2_ref[...]
    acc[0] += jnp.sum(e1 * e1) + jnp.sum(e2 * e2)
    acc[1] += jnp.sum(r1 * r1) + jnp.sum(r2 * r2)

    @pl.when(i == NBLK - 1)
    def _():
        o_ref[0] = acc[0]
        o_ref[1] = acc[1]


_norm_call = pl.pallas_call(
    _ssq_body,
    grid=(NBLK,),
    in_specs=[
        pl.BlockSpec((RB, D), lambda i: (i, 0)),
        pl.BlockSpec((RB, D), lambda i: (i + NBLK, 0)),
        pl.BlockSpec((RB, D), lambda i: (i, 0)),
        pl.BlockSpec((RB, D), lambda i: (i + NBLK, 0)),
    ],
    out_specs=pl.BlockSpec(memory_space=pltpu.SMEM),
    out_shape=jax.ShapeDtypeStruct((2,), jnp.float32),
    scratch_shapes=[pltpu.SMEM((2,), jnp.float32)],
)


def _loss_body(phi_ref, y_ref, ssq_ref, sce_ref, scr_ref, o_ref):
    z = -y_ref[...] * phi_ref[...]
    essq = ssq_ref[0] + jnp.sum(sce_ref[...])
    rssq = ssq_ref[1] + jnp.sum(scr_ref[...])
    o_ref[0] = (jnp.sum(jnp.log(1.0 + jnp.exp(z)))
                + LAMBDA_E * jnp.sqrt(essq) + LAMBDA_R * jnp.sqrt(rssq))


_loss_call = pl.pallas_call(
    _loss_body,
    in_specs=[
        pl.BlockSpec(memory_space=pltpu.VMEM),
        pl.BlockSpec(memory_space=pltpu.VMEM),
        pl.BlockSpec(memory_space=pltpu.SMEM),
        pl.BlockSpec(memory_space=pltpu.VMEM),
        pl.BlockSpec(memory_space=pltpu.VMEM),
    ],
    out_specs=pl.BlockSpec(memory_space=pltpu.SMEM),
    out_shape=jax.ShapeDtypeStruct((1,), jnp.float32),
)


def kernel(sample, Y, entity_embedding, relation_embedding):
    s32 = sample.astype(jnp.int32)
    hidx = s32[:, 0]
    ridx = s32[:, 1]
    tidx = s32[:, 2]
    phi, sce, scr = _phi_sc(entity_embedding, relation_embedding,
                            hidx, ridx, tidx)
    ssq = _norm_call(entity_embedding, entity_embedding,
                     relation_embedding, relation_embedding)
    loss = _loss_call(phi.reshape(128, 128), Y.reshape(128, 128), ssq,
                      sce.reshape(4, 128), scr.reshape(4, 128))
    return loss[0]


# speedup vs baseline: 1.0774x; 1.0774x over previous
import functools

import jax
import jax.numpy as jnp
from jax import lax
from jax.experimental import pallas as pl
from jax.experimental.pallas import tpu as pltpu
from jax.experimental.pallas import tpu_sc as plsc

D = 128
B = 16384
NE = 100000
NC = 2
NS = 16
NW = NC * NS
BPW = B // NW
CHUNK = 128
NCHUNK = BPW // CHUNK
LAMBDA_R = 0.05
LAMBDA_E = 0.01
SGNBIT = -2147483648


def _perm(x, idx):
    return jnp.take_along_axis(x, idx, axis=0)


def _rsqrt_nr(x):
    i = plsc.bitcast(x, jnp.int32)
    y = plsc.bitcast(jnp.int32(0x5F3759DF) - (i >> 1), jnp.float32)
    hx = 0.5 * x
    for _ in range(2):
        y = y * (1.5 - hx * y * y)
    return y


_sc_mesh = plsc.VectorSubcoreMesh(core_axis_name="c", subcore_axis_name="s")


@functools.partial(
    pl.kernel,
    mesh=_sc_mesh,
    compiler_params=pltpu.CompilerParams(needs_layout_passes=False),
    out_type=jax.ShapeDtypeStruct((B,), jnp.float32),
    scratch_types=[
        pltpu.VMEM((CHUNK,), jnp.int32),
        pltpu.VMEM((CHUNK,), jnp.int32),
        pltpu.VMEM((CHUNK,), jnp.int32),
        pltpu.VMEM((CHUNK, D), jnp.float32),
        pltpu.VMEM((CHUNK, D), jnp.float32),
        pltpu.VMEM((CHUNK, D), jnp.float32),
        pltpu.VMEM((CHUNK + 16,), jnp.float32),
        pltpu.SemaphoreType.DMA,
    ],
)
def _phi_sc(ent, rel, hidx, ridx, tidx, phi_out, hI, rI, tI, Hv, Rv, Tv,
            phiv, sem):
    wid = lax.axis_index("s") * NC + lax.axis_index("c")
    base = wid * BPW
    lanes = lax.iota(jnp.int32, 16)
    comp = lanes & 3
    px1 = lanes ^ 1
    px2 = lanes ^ 2
    px3 = lanes ^ 3
    px4 = lanes ^ 4
    px8 = lanes ^ 8
    grp = lanes & (-4)
    pb0 = grp
    pb1 = grp | 1
    pb2 = grp | 2
    pb3 = grp | 3
    zero = jnp.zeros((16,), jnp.int32)
    s0 = jnp.where(comp != 0, SGNBIT, zero)
    s1 = jnp.where(comp == 3, SGNBIT, zero)
    s2 = jnp.where(comp == 1, SGNBIT, zero)
    s3 = jnp.where(comp == 2, SGNBIT, zero)
    lane0 = lanes == 0

    def row_phi(row):
        acc = jnp.zeros((16,), jnp.float32)
        for v in range(D // 16):
            h = Hv[row, pl.ds(v * 16, 16)]
            r = Rv[row, pl.ds(v * 16, 16)]
            t = Tv[row, pl.ds(v * 16, 16)]
            r2 = r * r
            na = r2 + _perm(r2, px1)
            n = na + _perm(na, px2)
            rinv = _rsqrt_nr(n)
            hh = h * rinv
            ti = plsc.bitcast(t, jnp.int32)
            t0 = plsc.bitcast(_perm(ti, pb0) ^ s0, jnp.float32)
            t1 = plsc.bitcast(_perm(ti, pb1) ^ s1, jnp.float32)
            t2 = plsc.bitcast(_perm(ti, pb2) ^ s2, jnp.float32)
            t3 = plsc.bitcast(_perm(ti, pb3) ^ s3, jnp.float32)
            g = (r * t0 + _perm(r, px1) * t1
                 + _perm(r, px2) * t2 + _perm(r, px3) * t3)
            acc = acc + hh * g
        acc = acc + _perm(acc, px1)
        acc = acc + _perm(acc, px2)
        acc = acc + _perm(acc, px4)
        acc = acc + _perm(acc, px8)
        return acc

    def chunk_body(ci, _):
        off = base + ci * CHUNK
        iH = pltpu.async_copy(hidx.at[pl.ds(off, CHUNK)], hI, sem)
        iR = pltpu.async_copy(ridx.at[pl.ds(off, CHUNK)], rI, sem)
        iT = pltpu.async_copy(tidx.at[pl.ds(off, CHUNK)], tI, sem)
        iH.wait()
        iR.wait()
        iT.wait()
        cH = pltpu.async_copy(ent.at[hI], Hv, sem)
        cR = pltpu.async_copy(rel.at[rI], Rv, sem)
        cT = pltpu.async_copy(ent.at[tI], Tv, sem)
        cH.wait()
        cR.wait()
        cT.wait()

        def row_body(i, _):
            row = i * 4
            ps = [row_phi(row + j) for j in range(4)]
            for j in range(4):
                plsc.store_compressed(phiv.at[pl.ds(row + j, 16)], ps[j],
                                      mask=lane0)
            return _

        lax.fori_loop(0, CHUNK // 4, row_body, 0)
        pltpu.sync_copy(phiv.at[pl.ds(0, CHUNK)],
                        phi_out.at[pl.ds(off, CHUNK)])
        return _

    lax.fori_loop(0, NCHUNK, chunk_body, 0)


RB = 10000
NBLK = NE // RB


def _ssq_body(e_ref, r_ref, o_ref, acc):
    i = pl.program_id(0)

    @pl.when(i == 0)
    def _():
        acc[0] = 0.0
        acc[1] = 0.0

    e = e_ref[...]
    r = r_ref[...]
    acc[0] += jnp.sum(e * e)
    acc[1] += jnp.sum(r * r)

    @pl.when(i == NBLK - 1)
    def _():
        o_ref[0] = LAMBDA_E * jnp.sqrt(acc[0]) + LAMBDA_R * jnp.sqrt(acc[1])


_norm_call = pl.pallas_call(
    _ssq_body,
    grid=(NBLK,),
    in_specs=[
        pl.BlockSpec((RB, D), lambda i: (i, 0)),
        pl.BlockSpec((RB, D), lambda i: (i, 0)),
    ],
    out_specs=pl.BlockSpec(memory_space=pltpu.SMEM),
    out_shape=jax.ShapeDtypeStruct((1,), jnp.float32),
    scratch_shapes=[pltpu.SMEM((2,), jnp.float32)],
)


def _loss_body(phi_ref, y_ref, nt_ref, o_ref):
    z = -y_ref[...] * phi_ref[...]
    o_ref[0] = jnp.sum(jnp.log(1.0 + jnp.exp(z))) + nt_ref[0]


_loss_call = pl.pallas_call(
    _loss_body,
    in_specs=[
        pl.BlockSpec(memory_space=pltpu.VMEM),
        pl.BlockSpec(memory_space=pltpu.VMEM),
        pl.BlockSpec(memory_space=pltpu.SMEM),
    ],
    out_specs=pl.BlockSpec(memory_space=pltpu.SMEM),
    out_shape=jax.ShapeDtypeStruct((1,), jnp.float32),
)


def kernel(sample, Y, entity_embedding, relation_embedding):
    s32 = sample.astype(jnp.int32)
    hidx = s32[:, 0]
    ridx = s32[:, 1]
    tidx = s32[:, 2]
    phi = _phi_sc(entity_embedding, relation_embedding, hidx, ridx, tidx)
    nt = _norm_call(entity_embedding, relation_embedding)
    loss = _loss_call(phi.reshape(128, 128), Y.reshape(128, 128), nt)
    return loss[0]
